# Initial kernel scaffold; baseline (speedup 1.0000x reference)
import functools

import jax
import jax.numpy as jnp
from jax import lax
from jax.experimental import pallas as pl
from jax.experimental.pallas import tpu as pltpu
from jax.experimental.pallas import tpu_sc as plsc

B = 1024 * 26 * 20
D = 32
NC = 2
NS = 16
NW = NC * NS
B_PER_W = B // NW
CHUNK = 1664
NCHUNK = B_PER_W // CHUNK

_mesh = plsc.VectorSubcoreMesh(core_axis_name="c", subcore_axis_name="s")


@functools.partial(
    pl.kernel,
    mesh=_mesh,
    out_type=jax.ShapeDtypeStruct((B, D), jnp.float32),
    scratch_types=[
        pltpu.VMEM((CHUNK,), jnp.int32),
        pltpu.VMEM((CHUNK, D), jnp.float32),
        pltpu.SemaphoreType.DMA,
    ],
)
def _gather_kernel(table_hbm, idx_hbm, out_hbm, idx_v, rows_v, sem):
    wid = lax.axis_index("s") * NC + lax.axis_index("c")
    base0 = wid * B_PER_W

    def body(i, carry):
        base = base0 + i * CHUNK
        pltpu.sync_copy(idx_hbm.at[pl.ds(base, CHUNK)], idx_v)
        pltpu.async_copy(table_hbm.at[idx_v], rows_v, sem).wait()
        pltpu.sync_copy(rows_v, out_hbm.at[pl.ds(base, CHUNK)])
        return carry

    lax.fori_loop(0, NCHUNK, body, 0)


def kernel(x, W):
    xf = x.reshape(-1).astype(jnp.int32)
    out = _gather_kernel(W, xf)
    return out.reshape(x.shape + (D,))


# speedup vs baseline: 1.5147x; 1.5147x over previous
import functools

import jax
import jax.numpy as jnp
from jax import lax
from jax.experimental import pallas as pl
from jax.experimental.pallas import tpu as pltpu
from jax.experimental.pallas import tpu_sc as plsc

B = 1024 * 26 * 20
D = 32
NC = 2
NS = 16
NW = NC * NS
B_PER_W = B // NW
CHUNK = 1664
NCHUNK = B_PER_W // CHUNK

_mesh = plsc.VectorSubcoreMesh(core_axis_name="c", subcore_axis_name="s")


@functools.partial(
    pl.kernel,
    mesh=_mesh,
    out_type=jax.ShapeDtypeStruct((B, D), jnp.float32),
    scratch_types=[
        pltpu.VMEM((CHUNK,), jnp.int32),
        pltpu.VMEM((CHUNK, D), jnp.float32),
        pltpu.SemaphoreType.DMA,
    ],
    compiler_params=pltpu.CompilerParams(use_tc_tiling_on_sc=False),
)
def _gather_kernel(table_hbm, idx_hbm, out_hbm, idx_v, rows_v, sem):
    wid = lax.axis_index("s") * NC + lax.axis_index("c")
    base0 = wid * B_PER_W

    def body(i, carry):
        base = base0 + i * CHUNK
        pltpu.sync_copy(idx_hbm.at[pl.ds(base, CHUNK)], idx_v)
        pltpu.async_copy(table_hbm.at[idx_v], rows_v, sem).wait()
        pltpu.sync_copy(rows_v, out_hbm.at[pl.ds(base, CHUNK)])
        return carry

    lax.fori_loop(0, NCHUNK, body, 0)


def kernel(x, W):
    xf = x.reshape(-1).astype(jnp.int32)
    out = _gather_kernel(W, xf)
    return out.reshape(x.shape + (D,))


# speedup vs baseline: 1.5303x; 1.0103x over previous
import functools

import jax
import jax.numpy as jnp
from jax import lax
from jax.experimental import pallas as pl
from jax.experimental.pallas import tpu as pltpu
from jax.experimental.pallas import tpu_sc as plsc

B = 1024 * 26 * 20
D = 32
NC = 2
NS = 16
NW = NC * NS
B_PER_W = B // NW
CHUNK = 1664
NCHUNK = B_PER_W // CHUNK

_mesh = plsc.VectorSubcoreMesh(core_axis_name="c", subcore_axis_name="s")


NBUF = 2


@functools.partial(
    pl.kernel,
    mesh=_mesh,
    out_type=jax.ShapeDtypeStruct((B, D), jnp.float32),
    scratch_types=[
        pltpu.VMEM((NBUF, CHUNK), jnp.int32),
        pltpu.VMEM((NBUF, CHUNK, D), jnp.float32),
        pltpu.SemaphoreType.DMA,
        pltpu.SemaphoreType.DMA,
    ],
    compiler_params=pltpu.CompilerParams(use_tc_tiling_on_sc=False),
)
def _gather_kernel(table_hbm, idx_hbm, out_hbm, idx_v, rows_v, gsem, ssem):
    wid = lax.axis_index("s") * NC + lax.axis_index("c")
    base0 = wid * B_PER_W

    gathers = [None] * NCHUNK
    stores = [None] * NCHUNK
    for i in range(NBUF):
        b = i % NBUF
        pltpu.sync_copy(idx_hbm.at[pl.ds(base0 + i * CHUNK, CHUNK)], idx_v.at[b])
        gathers[i] = pltpu.async_copy(table_hbm.at[idx_v.at[b]], rows_v.at[b], gsem)
    for i in range(NCHUNK):
        b = i % NBUF
        gathers[i].wait()
        stores[i] = pltpu.async_copy(
            rows_v.at[b], out_hbm.at[pl.ds(base0 + i * CHUNK, CHUNK)], ssem
        )
        j = i + NBUF
        if j < NCHUNK:
            pltpu.sync_copy(idx_hbm.at[pl.ds(base0 + j * CHUNK, CHUNK)], idx_v.at[b])
            stores[i].wait()
            gathers[j] = pltpu.async_copy(table_hbm.at[idx_v.at[b]], rows_v.at[b], gsem)
    for i in range(NCHUNK - NBUF, NCHUNK):
        stores[i].wait()


def kernel(x, W):
    xf = x.reshape(-1).astype(jnp.int32)
    out = _gather_kernel(W, xf)
    return out.reshape(x.shape + (D,))


# speedup vs baseline: 1.5324x; 1.0013x over previous
import functools

import jax
import jax.numpy as jnp
from jax import lax
from jax.experimental import pallas as pl
from jax.experimental.pallas import tpu as pltpu
from jax.experimental.pallas import tpu_sc as plsc

B = 1024 * 26 * 20
D = 32
NC = 2
NS = 16
NW = NC * NS
B_PER_W = B // NW
CHUNK = 1664
NCHUNK = B_PER_W // CHUNK

_mesh = plsc.VectorSubcoreMesh(core_axis_name="c", subcore_axis_name="s")


NBUF = 2
KSUB = 4
SUB = CHUNK // KSUB


@functools.partial(
    pl.kernel,
    mesh=_mesh,
    out_type=jax.ShapeDtypeStruct((B, D), jnp.float32),
    scratch_types=[
        pltpu.VMEM((NBUF, CHUNK), jnp.int32),
        pltpu.VMEM((NBUF, CHUNK, D), jnp.float32),
        pltpu.SemaphoreType.DMA,
        pltpu.SemaphoreType.DMA,
    ],
    compiler_params=pltpu.CompilerParams(use_tc_tiling_on_sc=False),
)
def _gather_kernel(table_hbm, idx_hbm, out_hbm, idx_v, rows_v, gsem, ssem):
    wid = lax.axis_index("s") * NC + lax.axis_index("c")
    base0 = wid * B_PER_W

    def fire_gathers(b):
        return [
            pltpu.async_copy(
                table_hbm.at[idx_v.at[b, pl.ds(q * SUB, SUB)]],
                rows_v.at[b, pl.ds(q * SUB, SUB)],
                gsem,
            )
            for q in range(KSUB)
        ]

    gathers = [None] * NCHUNK
    stores = [None] * NCHUNK
    for i in range(NBUF):
        b = i % NBUF
        pltpu.sync_copy(idx_hbm.at[pl.ds(base0 + i * CHUNK, CHUNK)], idx_v.at[b])
        gathers[i] = fire_gathers(b)
    for i in range(NCHUNK):
        b = i % NBUF
        for g in gathers[i]:
            g.wait()
        stores[i] = pltpu.async_copy(
            rows_v.at[b], out_hbm.at[pl.ds(base0 + i * CHUNK, CHUNK)], ssem
        )
        j = i + NBUF
        if j < NCHUNK:
            pltpu.sync_copy(idx_hbm.at[pl.ds(base0 + j * CHUNK, CHUNK)], idx_v.at[b])
            stores[i].wait()
            gathers[j] = fire_gathers(b)
    for i in range(NCHUNK - NBUF, NCHUNK):
        stores[i].wait()


def kernel(x, W):
    xf = x.reshape(-1).astype(jnp.int32)
    out = _gather_kernel(W, xf)
    return out.reshape(x.shape + (D,))
